# baseline (speedup 1.0000x reference)
import functools

import jax
import jax.numpy as jnp
from jax import lax
from jax.experimental import pallas as pl
from jax.experimental.pallas import tpu as pltpu
from jax.experimental.pallas import tpu_sc as plsc

B_, L1_, L2_, D_ = 16, 512, 2048, 768
K_ = 256
_NEG = -10000000000.0


def _tc_body(off, g1_ref, g2_ref, mrow_ref, mcol_ref, gidx_ref, xmask_ref):
    b = pl.program_id(0) + off
    g1 = g1_ref[0]
    g2 = g2_ref[0]
    mrow = mrow_ref[0]
    mcol = mcol_ref[0]

    e = lax.dot_general(g1, g2, (((1,), (1,)), ((), ())),
                        preferred_element_type=jnp.float32)
    e = jnp.where(mrow > 0.5, e, _NEG)
    m = jnp.max(e, axis=1, keepdims=True)
    p = jnp.exp(e - m)
    s = jnp.sum(p, axis=1, keepdims=True)
    a = p / s
    w_row = jnp.sum(a, axis=0, keepdims=True)

    w8 = jnp.broadcast_to(w_row, (8, L2_))
    eye8 = (lax.broadcasted_iota(jnp.int32, (8, 8), 0)
            == lax.broadcasted_iota(jnp.int32, (8, 8), 1)).astype(jnp.float32)
    wT = lax.dot_general(w8, eye8, (((0,), (0,)), ((), ())),
                         precision=lax.Precision.HIGHEST,
                         preferred_element_type=jnp.float32)
    w_col = wT[:, 0:1]

    CH = 512
    k_row = lax.broadcasted_iota(jnp.int32, (CH, L2_), 1)
    r_iota = lax.broadcasted_iota(jnp.int32, (CH, K_), 1)
    jg = lax.broadcasted_iota(jnp.int32, (CH, K_), 0)
    idx_sum = jnp.zeros((1, K_), jnp.int32)
    msk_sum = jnp.zeros((1, K_), jnp.float32)
    for c in range(L2_ // CH):
        wj = w_col[c * CH:(c + 1) * CH, :]
        j_col = c * CH + lax.broadcasted_iota(jnp.int32, (CH, L2_), 0)
        before = (w_row > wj) | ((w_row == wj) & (k_row < j_col))
        cnt = jnp.sum(before.astype(jnp.int32), axis=1, keepdims=True)
        oh = (cnt == r_iota)
        idx_sum = idx_sum + jnp.sum(jnp.where(oh, c * CH + jg, 0),
                                    axis=0, keepdims=True)
        mj = mcol[c * CH:(c + 1) * CH, :]
        msk_sum = msk_sum + jnp.sum(jnp.where(oh, mj, 0.0),
                                    axis=0, keepdims=True)

    gidx_ref[...] = (idx_sum + b * L2_).reshape(1, 1, K_)
    xmask_ref[...] = msk_sum.reshape(1, 1, K_)


def _tc_topk(g_r1, g_r2, mask_row, mask_col, off=0, interpret=False):
    nb = g_r1.shape[0]
    return pl.pallas_call(
        functools.partial(_tc_body, off),
        grid=(nb,),
        in_specs=[
            pl.BlockSpec((1, L1_, D_), lambda b: (b, 0, 0)),
            pl.BlockSpec((1, L2_, D_), lambda b: (b, 0, 0)),
            pl.BlockSpec((1, 1, L2_), lambda b: (b, 0, 0)),
            pl.BlockSpec((1, L2_, 1), lambda b: (b, 0, 0)),
        ],
        out_specs=[
            pl.BlockSpec((1, 1, K_), lambda b: (b, 0, 0)),
            pl.BlockSpec((1, 1, K_), lambda b: (b, 0, 0)),
        ],
        out_shape=[
            jax.ShapeDtypeStruct((nb, 1, K_), jnp.int32),
            jax.ShapeDtypeStruct((nb, 1, K_), jnp.float32),
        ],
        interpret=interpret,
    )(g_r1, g_r2, mask_row, mask_col)


_NC, _NS = 2, 16
_NW = _NC * _NS
_ROWS = B_ * K_
_BPW = _ROWS // _NW


@functools.lru_cache(maxsize=4)
def _sc_gather_fn(nrows):
    bpw = nrows // _NW

    @functools.partial(
        pl.kernel,
        mesh=plsc.VectorSubcoreMesh(core_axis_name="c", subcore_axis_name="s"),
        out_type=jax.ShapeDtypeStruct((nrows, D_), jnp.float32),
        scratch_types=[
            pltpu.VMEM((bpw,), jnp.int32),
            pltpu.VMEM((bpw, D_), jnp.float32),
            pltpu.SemaphoreType.DMA,
        ],
    )
    def _sc_gather(table_hbm, idx_hbm, out_hbm, idx_v, rows_v, sem):
        wid = lax.axis_index("s") * _NC + lax.axis_index("c")
        base = wid * bpw
        pltpu.sync_copy(idx_hbm.at[pl.ds(base, bpw)], idx_v)
        pltpu.async_copy(table_hbm.at[idx_v], rows_v, sem).wait()
        pltpu.sync_copy(rows_v, out_hbm.at[pl.ds(base, bpw)])

    return _sc_gather


def kernel(g_r1, g_r2, g_r1_mask, g_r2_mask):
    m2 = g_r2_mask.reshape(B_, L2_).astype(jnp.float32)
    mrow = m2.reshape(B_, 1, L2_)
    mcol = m2.reshape(B_, L2_, 1)
    table = g_r2.reshape(B_ * L2_, D_)
    H = B_ // 2
    xs, xms = [], []
    for h in range(2):
        sl = slice(h * H, (h + 1) * H)
        gidx, xmask = _tc_topk(g_r1[sl], g_r2[sl], mrow[sl], mcol[sl],
                               off=h * H)
        xs.append(_sc_gather_fn(H * K_)(table, gidx.reshape(H * K_)))
        xms.append(xmask)
    x = jnp.concatenate(xs, axis=0).reshape(B_, K_, D_)
    xmask_all = jnp.concatenate(xms, axis=0)
    x_mask = (xmask_all.reshape(B_, 1, 1, K_) > 0.5)
    return (x, x_mask)


# speedup vs baseline: 1.5026x; 1.5026x over previous
import functools

import jax
import jax.numpy as jnp
from jax import lax
from jax.experimental import pallas as pl
from jax.experimental.pallas import tpu as pltpu
from jax.experimental.pallas import tpu_sc as plsc

B_, L1_, L2_, D_ = 16, 512, 2048, 768
K_ = 256
_NEG = -10000000000.0


def _tc_body(off, g1_ref, g2_ref, mrow_ref, mcol_ref, gidx_ref, xmask_ref):
    b = pl.program_id(0) + off
    g1 = g1_ref[0]
    g2 = g2_ref[0]
    mrow = mrow_ref[0]
    mcol = mcol_ref[0]

    e = lax.dot_general(g1, g2, (((1,), (1,)), ((), ())),
                        preferred_element_type=jnp.float32)
    e = jnp.where(mrow > 0.5, e, _NEG)
    m = jnp.max(e, axis=1, keepdims=True)
    p = jnp.exp(e - m)
    s = jnp.sum(p, axis=1, keepdims=True)
    a = p / s
    w_row = jnp.sum(a, axis=0, keepdims=True)

    w8 = jnp.broadcast_to(w_row, (8, L2_))
    eye8 = (lax.broadcasted_iota(jnp.int32, (8, 8), 0)
            == lax.broadcasted_iota(jnp.int32, (8, 8), 1)).astype(jnp.float32)
    wT = lax.dot_general(w8, eye8, (((0,), (0,)), ((), ())),
                         precision=lax.Precision.HIGHEST,
                         preferred_element_type=jnp.float32)
    w_col = wT[:, 0:1]

    CH = 512
    k_row = lax.broadcasted_iota(jnp.int32, (CH, L2_), 1)
    r_iota = lax.broadcasted_iota(jnp.int32, (CH, K_), 1)
    jg = lax.broadcasted_iota(jnp.int32, (CH, K_), 0)
    idx_sum = jnp.zeros((1, K_), jnp.int32)
    msk_sum = jnp.zeros((1, K_), jnp.float32)
    for c in range(L2_ // CH):
        wj = w_col[c * CH:(c + 1) * CH, :]
        j_col = c * CH + lax.broadcasted_iota(jnp.int32, (CH, L2_), 0)
        before = (w_row > wj) | ((w_row == wj) & (k_row < j_col))
        cnt = jnp.sum(before.astype(jnp.int32), axis=1, keepdims=True)
        oh = (cnt == r_iota)
        idx_sum = idx_sum + jnp.sum(jnp.where(oh, c * CH + jg, 0),
                                    axis=0, keepdims=True)
        mj = mcol[c * CH:(c + 1) * CH, :]
        msk_sum = msk_sum + jnp.sum(jnp.where(oh, mj, 0.0),
                                    axis=0, keepdims=True)

    gidx_ref[...] = (idx_sum + b * L2_).reshape(1, 1, K_)
    xmask_ref[...] = msk_sum.reshape(1, 1, K_)


def _tc_topk(g_r1, g_r2, mask_row, mask_col, off=0, nb=B_, interpret=False):
    return pl.pallas_call(
        functools.partial(_tc_body, off),
        grid=(nb,),
        in_specs=[
            pl.BlockSpec((1, L1_, D_), lambda b: (b + off, 0, 0)),
            pl.BlockSpec((1, L2_, D_), lambda b: (b + off, 0, 0)),
            pl.BlockSpec((1, 1, L2_), lambda b: (b + off, 0, 0)),
            pl.BlockSpec((1, L2_, 1), lambda b: (b + off, 0, 0)),
        ],
        out_specs=[
            pl.BlockSpec((1, 1, K_), lambda b: (b, 0, 0)),
            pl.BlockSpec((1, 1, K_), lambda b: (b, 0, 0)),
        ],
        out_shape=[
            jax.ShapeDtypeStruct((nb, 1, K_), jnp.int32),
            jax.ShapeDtypeStruct((nb, 1, K_), jnp.float32),
        ],
        interpret=interpret,
    )(g_r1, g_r2, mask_row, mask_col)


_NC, _NS = 2, 16
_NW = _NC * _NS
_ROWS = B_ * K_
_BPW = _ROWS // _NW


@functools.lru_cache(maxsize=4)
def _sc_gather_fn(nrows):
    bpw = nrows // _NW

    @functools.partial(
        pl.kernel,
        mesh=plsc.VectorSubcoreMesh(core_axis_name="c", subcore_axis_name="s"),
        out_type=jax.ShapeDtypeStruct((nrows, D_), jnp.float32),
        scratch_types=[
            pltpu.VMEM((bpw,), jnp.int32),
            pltpu.VMEM((bpw, D_), jnp.float32),
            pltpu.SemaphoreType.DMA,
        ],
    )
    def _sc_gather(table_hbm, idx_hbm, out_hbm, idx_v, rows_v, sem):
        wid = lax.axis_index("s") * _NC + lax.axis_index("c")
        base = wid * bpw
        pltpu.sync_copy(idx_hbm.at[pl.ds(base, bpw)], idx_v)
        pltpu.async_copy(table_hbm.at[idx_v], rows_v, sem).wait()
        pltpu.sync_copy(rows_v, out_hbm.at[pl.ds(base, bpw)])

    return _sc_gather


def kernel(g_r1, g_r2, g_r1_mask, g_r2_mask):
    m2 = g_r2_mask.reshape(B_, L2_).astype(jnp.float32)
    mrow = m2.reshape(B_, 1, L2_)
    mcol = m2.reshape(B_, L2_, 1)
    table = g_r2.reshape(B_ * L2_, D_)
    H = B_ // 2
    xs, xms = [], []
    for h in range(2):
        gidx, xmask = _tc_topk(g_r1, g_r2, mrow, mcol, off=h * H, nb=H)
        xs.append(_sc_gather_fn(H * K_)(table, gidx.reshape(H * K_)))
        xms.append(xmask)
    x = jnp.concatenate(xs, axis=0).reshape(B_, K_, D_)
    xmask_all = jnp.concatenate(xms, axis=0)
    x_mask = (xmask_all.reshape(B_, 1, 1, K_) > 0.5)
    return (x, x_mask)


# speedup vs baseline: 1.6213x; 1.0790x over previous
import functools

import jax
import jax.numpy as jnp
from jax import lax
from jax.experimental import pallas as pl
from jax.experimental.pallas import tpu as pltpu
from jax.experimental.pallas import tpu_sc as plsc

B_, L1_, L2_, D_ = 16, 512, 2048, 768
K_ = 256
_NEG = -10000000000.0


def _tc_body(off, g1_ref, g2_ref, mrow_ref, mcol_ref, gidx_ref, xmask_ref):
    b = pl.program_id(0) + off
    g1 = g1_ref[0]
    g2 = g2_ref[0]
    mrow = mrow_ref[0]
    mcol = mcol_ref[0]

    e = lax.dot_general(g1, g2, (((1,), (1,)), ((), ())),
                        preferred_element_type=jnp.float32)
    e = jnp.where(mrow > 0.5, e, _NEG)
    m = jnp.max(e, axis=1, keepdims=True)
    p = jnp.exp(e - m)
    s = jnp.sum(p, axis=1, keepdims=True)
    a = p / s
    w_row = jnp.sum(a, axis=0, keepdims=True)

    w8 = jnp.broadcast_to(w_row, (8, L2_))
    eye8 = (lax.broadcasted_iota(jnp.int32, (8, 8), 0)
            == lax.broadcasted_iota(jnp.int32, (8, 8), 1)).astype(jnp.float32)
    wT = lax.dot_general(w8, eye8, (((0,), (0,)), ((), ())),
                         precision=lax.Precision.HIGHEST,
                         preferred_element_type=jnp.float32)
    w_col = wT[:, 0:1]

    CH = 512
    TL = 128
    DIAG = CH // TL
    k_loc0 = lax.broadcasted_iota(jnp.int32, (CH, TL), 1)
    j_loc = lax.broadcasted_iota(jnp.int32, (CH, TL), 0)
    r_iota = lax.broadcasted_iota(jnp.int32, (CH, K_), 1)
    jg = lax.broadcasted_iota(jnp.int32, (CH, K_), 0)
    idx_sum = jnp.zeros((1, K_), jnp.int32)
    msk_sum = jnp.zeros((1, K_), jnp.float32)
    for c in range(L2_ // CH):
        wj = w_col[c * CH:(c + 1) * CH, :]
        acc = jnp.zeros((CH, TL), jnp.int32)
        for t in range(L2_ // TL):
            wk = w_row[:, t * TL:(t + 1) * TL]
            if t < c * DIAG:
                hit = (wk >= wj)
            elif t >= (c + 1) * DIAG:
                hit = (wk > wj)
            else:
                less = (k_loc0 + (t * TL - c * CH)) < j_loc
                hit = (wk > wj) | ((wk == wj) & less)
            acc = acc + hit.astype(jnp.int32)
        cnt = jnp.sum(acc, axis=1, keepdims=True)
        oh = (cnt == r_iota)
        idx_sum = idx_sum + jnp.sum(jnp.where(oh, c * CH + jg, 0),
                                    axis=0, keepdims=True)
        mj = mcol[c * CH:(c + 1) * CH, :]
        msk_sum = msk_sum + jnp.sum(jnp.where(oh, mj, 0.0),
                                    axis=0, keepdims=True)

    gidx_ref[...] = (idx_sum + b * L2_).reshape(1, 1, K_)
    xmask_ref[...] = msk_sum.reshape(1, 1, K_)


def _tc_topk(g_r1, g_r2, mask_row, mask_col, off=0, nb=B_, interpret=False):
    return pl.pallas_call(
        functools.partial(_tc_body, off),
        grid=(nb,),
        in_specs=[
            pl.BlockSpec((1, L1_, D_), lambda b: (b + off, 0, 0)),
            pl.BlockSpec((1, L2_, D_), lambda b: (b + off, 0, 0)),
            pl.BlockSpec((1, 1, L2_), lambda b: (b + off, 0, 0)),
            pl.BlockSpec((1, L2_, 1), lambda b: (b + off, 0, 0)),
        ],
        out_specs=[
            pl.BlockSpec((1, 1, K_), lambda b: (b, 0, 0)),
            pl.BlockSpec((1, 1, K_), lambda b: (b, 0, 0)),
        ],
        out_shape=[
            jax.ShapeDtypeStruct((nb, 1, K_), jnp.int32),
            jax.ShapeDtypeStruct((nb, 1, K_), jnp.float32),
        ],
        interpret=interpret,
    )(g_r1, g_r2, mask_row, mask_col)


_NC, _NS = 2, 16
_NW = _NC * _NS
_ROWS = B_ * K_
_BPW = _ROWS // _NW


@functools.lru_cache(maxsize=4)
def _sc_gather_fn(nrows):
    bpw = nrows // _NW

    @functools.partial(
        pl.kernel,
        mesh=plsc.VectorSubcoreMesh(core_axis_name="c", subcore_axis_name="s"),
        out_type=jax.ShapeDtypeStruct((nrows, D_), jnp.float32),
        scratch_types=[
            pltpu.VMEM((bpw,), jnp.int32),
            pltpu.VMEM((bpw, D_), jnp.float32),
            pltpu.SemaphoreType.DMA,
        ],
    )
    def _sc_gather(table_hbm, idx_hbm, out_hbm, idx_v, rows_v, sem):
        wid = lax.axis_index("s") * _NC + lax.axis_index("c")
        base = wid * bpw
        pltpu.sync_copy(idx_hbm.at[pl.ds(base, bpw)], idx_v)
        pltpu.async_copy(table_hbm.at[idx_v], rows_v, sem).wait()
        pltpu.sync_copy(rows_v, out_hbm.at[pl.ds(base, bpw)])

    return _sc_gather


def kernel(g_r1, g_r2, g_r1_mask, g_r2_mask):
    m2 = g_r2_mask.reshape(B_, L2_).astype(jnp.float32)
    mrow = m2.reshape(B_, 1, L2_)
    mcol = m2.reshape(B_, L2_, 1)
    table = g_r2.reshape(B_ * L2_, D_)
    gidx, xmask = _tc_topk(g_r1, g_r2, mrow, mcol)
    xflat = _sc_gather_fn(_ROWS)(table, gidx.reshape(_ROWS))
    x = xflat.reshape(B_, K_, D_)
    x_mask = (xmask.reshape(B_, 1, 1, K_) > 0.5)
    return (x, x_mask)


# speedup vs baseline: 1.8600x; 1.1472x over previous
import functools

import jax
import jax.numpy as jnp
from jax import lax
from jax.experimental import pallas as pl
from jax.experimental.pallas import tpu as pltpu
from jax.experimental.pallas import tpu_sc as plsc

B_, L1_, L2_, D_ = 16, 512, 2048, 768
K_ = 256
_NEG = -10000000000.0


def _tc_body(off, g1_ref, g2_ref, mrow_ref, mcol_ref, gidx_ref, xmask_ref):
    b = pl.program_id(0) + off
    g1 = g1_ref[0]
    g2 = g2_ref[0]
    mrow = mrow_ref[0]
    mcol = mcol_ref[0]

    e = lax.dot_general(g1, g2, (((1,), (1,)), ((), ())),
                        preferred_element_type=jnp.float32)
    e = jnp.where(mrow > 0.5, e, _NEG)
    m = jnp.max(e, axis=1, keepdims=True)
    p = jnp.exp(e - m)
    s = jnp.sum(p, axis=1, keepdims=True)
    a = p / s
    w_row = jnp.sum(a, axis=0, keepdims=True)

    w8 = jnp.broadcast_to(w_row, (8, L2_))
    wT = jnp.transpose(w8)
    w_col = wT[:, 0:1]

    CH = 512
    TL = 128
    DIAG = CH // TL
    k_loc0 = lax.broadcasted_iota(jnp.int32, (CH, TL), 1)
    j_loc = lax.broadcasted_iota(jnp.int32, (CH, TL), 0)
    r_iota = lax.broadcasted_iota(jnp.int32, (CH, K_), 1)
    jg = lax.broadcasted_iota(jnp.int32, (CH, K_), 0)
    idx_sum = jnp.zeros((1, K_), jnp.int32)
    msk_sum = jnp.zeros((1, K_), jnp.float32)
    for c in range(L2_ // CH):
        wj = w_col[c * CH:(c + 1) * CH, :]
        acc = jnp.zeros((CH, TL), jnp.int32)
        for t in range(L2_ // TL):
            wk = w_row[:, t * TL:(t + 1) * TL]
            if t < c * DIAG:
                hit = (wk >= wj)
            elif t >= (c + 1) * DIAG:
                hit = (wk > wj)
            else:
                less = (k_loc0 + (t * TL - c * CH)) < j_loc
                hit = (wk > wj) | ((wk == wj) & less)
            acc = acc + hit.astype(jnp.int32)
        cnt = jnp.sum(acc, axis=1, keepdims=True)
        oh = (cnt == r_iota)
        idx_sum = idx_sum + jnp.sum(jnp.where(oh, c * CH + jg, 0),
                                    axis=0, keepdims=True)
        mj = mcol[c * CH:(c + 1) * CH, :]
        msk_sum = msk_sum + jnp.sum(jnp.where(oh, mj, 0.0),
                                    axis=0, keepdims=True)

    gidx_ref[...] = (idx_sum + b * L2_).reshape(1, 1, K_)
    xmask_ref[...] = msk_sum.reshape(1, 1, K_)


def _tc_topk(g_r1, g_r2, mask_row, mask_col, off=0, nb=B_, interpret=False):
    return pl.pallas_call(
        functools.partial(_tc_body, off),
        grid=(nb,),
        in_specs=[
            pl.BlockSpec((1, L1_, D_), lambda b: (b + off, 0, 0)),
            pl.BlockSpec((1, L2_, D_), lambda b: (b + off, 0, 0)),
            pl.BlockSpec((1, 1, L2_), lambda b: (b + off, 0, 0)),
            pl.BlockSpec((1, L2_, 1), lambda b: (b + off, 0, 0)),
        ],
        out_specs=[
            pl.BlockSpec((1, 1, K_), lambda b: (b, 0, 0)),
            pl.BlockSpec((1, 1, K_), lambda b: (b, 0, 0)),
        ],
        out_shape=[
            jax.ShapeDtypeStruct((nb, 1, K_), jnp.int32),
            jax.ShapeDtypeStruct((nb, 1, K_), jnp.float32),
        ],
        interpret=interpret,
    )(g_r1, g_r2, mask_row, mask_col)


_NC, _NS = 2, 16
_NW = _NC * _NS
_ROWS = B_ * K_
_BPW = _ROWS // _NW


@functools.lru_cache(maxsize=4)
def _sc_gather_fn(nrows):
    bpw = nrows // _NW

    @functools.partial(
        pl.kernel,
        mesh=plsc.VectorSubcoreMesh(core_axis_name="c", subcore_axis_name="s"),
        out_type=jax.ShapeDtypeStruct((nrows, D_), jnp.float32),
        scratch_types=[
            pltpu.VMEM((bpw,), jnp.int32),
            pltpu.VMEM((bpw, D_), jnp.float32),
            pltpu.SemaphoreType.DMA,
        ],
    )
    def _sc_gather(table_hbm, idx_hbm, out_hbm, idx_v, rows_v, sem):
        wid = lax.axis_index("s") * _NC + lax.axis_index("c")
        base = wid * bpw
        pltpu.sync_copy(idx_hbm.at[pl.ds(base, bpw)], idx_v)
        pltpu.async_copy(table_hbm.at[idx_v], rows_v, sem).wait()
        pltpu.sync_copy(rows_v, out_hbm.at[pl.ds(base, bpw)])

    return _sc_gather


def kernel(g_r1, g_r2, g_r1_mask, g_r2_mask):
    m2 = g_r2_mask.reshape(B_, L2_).astype(jnp.float32)
    mrow = m2.reshape(B_, 1, L2_)
    mcol = m2.reshape(B_, L2_, 1)
    table = g_r2.reshape(B_ * L2_, D_)
    gidx, xmask = _tc_topk(g_r1, g_r2, mrow, mcol)
    xflat = _sc_gather_fn(_ROWS)(table, gidx.reshape(_ROWS))
    x = xflat.reshape(B_, K_, D_)
    x_mask = (xmask.reshape(B_, 1, 1, K_) > 0.5)
    return (x, x_mask)


# speedup vs baseline: 1.8738x; 1.0074x over previous
import functools

import jax
import jax.numpy as jnp
from jax import lax
from jax.experimental import pallas as pl
from jax.experimental.pallas import tpu as pltpu
from jax.experimental.pallas import tpu_sc as plsc

B_, L1_, L2_, D_ = 16, 512, 2048, 768
K_ = 256
_NEG = -10000000000.0


def _tc_body(off, g1_ref, g2_ref, mrow_ref, mcol_ref, gidx_ref, xmask_ref):
    b = pl.program_id(0) + off
    g1 = g1_ref[0]
    g2 = g2_ref[0]
    mrow = mrow_ref[0]
    mcol = mcol_ref[0]

    e = lax.dot_general(g1, g2, (((1,), (1,)), ((), ())),
                        preferred_element_type=jnp.float32)
    e = jnp.where(mrow > 0.5, e, _NEG)
    m = jnp.max(e, axis=1, keepdims=True)
    p = jnp.exp(e - m)
    s = jnp.sum(p, axis=1, keepdims=True)
    a = p / s
    w_row = jnp.sum(a, axis=0, keepdims=True)

    w8 = jnp.broadcast_to(w_row, (8, L2_))
    wT = jnp.transpose(w8)
    w_col = wT[:, 0:1]

    CH = 256
    TL = 128
    DIAG = CH // TL
    k_loc0 = lax.broadcasted_iota(jnp.int32, (CH, TL), 1)
    j_loc = lax.broadcasted_iota(jnp.int32, (CH, TL), 0)
    less_d = [(k_loc0 + d * TL) < j_loc for d in range(DIAG)]
    r_iota = lax.broadcasted_iota(jnp.int32, (CH, K_), 1)
    jg = lax.broadcasted_iota(jnp.int32, (CH, K_), 0)
    idx_sum = jnp.zeros((1, K_), jnp.int32)
    msk_sum = jnp.zeros((1, K_), jnp.float32)
    for c in range(L2_ // CH):
        wj = w_col[c * CH:(c + 1) * CH, :]
        acc = jnp.zeros((CH, TL), jnp.int32)
        for t in range(L2_ // TL):
            wk = w_row[:, t * TL:(t + 1) * TL]
            if t < c * DIAG:
                hit = (wk >= wj)
            elif t >= (c + 1) * DIAG:
                hit = (wk > wj)
            else:
                hit = (wk > wj) | ((wk == wj) & less_d[t - c * DIAG])
            acc = acc + hit.astype(jnp.int32)
        cnt = jnp.sum(acc, axis=1, keepdims=True)
        oh = (cnt == r_iota)
        idx_sum = idx_sum + jnp.sum(jnp.where(oh, c * CH + jg, 0),
                                    axis=0, keepdims=True)
        mj = mcol[c * CH:(c + 1) * CH, :]
        msk_sum = msk_sum + jnp.sum(jnp.where(oh, mj, 0.0),
                                    axis=0, keepdims=True)

    gidx_ref[...] = (idx_sum + b * L2_).reshape(1, 1, K_)
    xmask_ref[...] = msk_sum.reshape(1, 1, K_)


def _tc_topk(g_r1, g_r2, mask_row, mask_col, off=0, nb=B_, interpret=False):
    return pl.pallas_call(
        functools.partial(_tc_body, off),
        grid=(nb,),
        in_specs=[
            pl.BlockSpec((1, L1_, D_), lambda b: (b + off, 0, 0)),
            pl.BlockSpec((1, L2_, D_), lambda b: (b + off, 0, 0)),
            pl.BlockSpec((1, 1, L2_), lambda b: (b + off, 0, 0)),
            pl.BlockSpec((1, L2_, 1), lambda b: (b + off, 0, 0)),
        ],
        out_specs=[
            pl.BlockSpec((1, 1, K_), lambda b: (b, 0, 0)),
            pl.BlockSpec((1, 1, K_), lambda b: (b, 0, 0)),
        ],
        out_shape=[
            jax.ShapeDtypeStruct((nb, 1, K_), jnp.int32),
            jax.ShapeDtypeStruct((nb, 1, K_), jnp.float32),
        ],
        interpret=interpret,
    )(g_r1, g_r2, mask_row, mask_col)


_NC, _NS = 2, 16
_NW = _NC * _NS
_ROWS = B_ * K_
_BPW = _ROWS // _NW


@functools.lru_cache(maxsize=4)
def _sc_gather_fn(nrows):
    bpw = nrows // _NW

    @functools.partial(
        pl.kernel,
        mesh=plsc.VectorSubcoreMesh(core_axis_name="c", subcore_axis_name="s"),
        out_type=jax.ShapeDtypeStruct((nrows, D_), jnp.float32),
        scratch_types=[
            pltpu.VMEM((bpw,), jnp.int32),
            pltpu.VMEM((bpw, D_), jnp.float32),
            pltpu.SemaphoreType.DMA,
        ],
    )
    def _sc_gather(table_hbm, idx_hbm, out_hbm, idx_v, rows_v, sem):
        wid = lax.axis_index("s") * _NC + lax.axis_index("c")
        base = wid * bpw
        pltpu.sync_copy(idx_hbm.at[pl.ds(base, bpw)], idx_v)
        pltpu.async_copy(table_hbm.at[idx_v], rows_v, sem).wait()
        pltpu.sync_copy(rows_v, out_hbm.at[pl.ds(base, bpw)])

    return _sc_gather


def kernel(g_r1, g_r2, g_r1_mask, g_r2_mask):
    m2 = g_r2_mask.reshape(B_, L2_).astype(jnp.float32)
    mrow = m2.reshape(B_, 1, L2_)
    mcol = m2.reshape(B_, L2_, 1)
    table = g_r2.reshape(B_ * L2_, D_)
    gidx, xmask = _tc_topk(g_r1, g_r2, mrow, mcol)
    xflat = _sc_gather_fn(_ROWS)(table, gidx.reshape(_ROWS))
    x = xflat.reshape(B_, K_, D_)
    x_mask = (xmask.reshape(B_, 1, 1, K_) > 0.5)
    return (x, x_mask)


# speedup vs baseline: 2.0066x; 1.0709x over previous
import functools

import jax
import jax.numpy as jnp
from jax import lax
from jax.experimental import pallas as pl
from jax.experimental.pallas import tpu as pltpu
from jax.experimental.pallas import tpu_sc as plsc

B_, L1_, L2_, D_ = 16, 512, 2048, 768
K_ = 256
_NEG = -10000000000.0


def _tc_body(off, g1_ref, g2_ref, mrow_ref, gidx_ref, xmask_ref):
    b = pl.program_id(0) + off
    g1 = g1_ref[0]
    g2 = g2_ref[0]
    mrow = mrow_ref[0]
    mcol = jnp.transpose(jnp.broadcast_to(mrow, (8, L2_)))[:, 0:1]

    e = lax.dot_general(g1, g2, (((1,), (1,)), ((), ())),
                        preferred_element_type=jnp.float32)
    e = jnp.where(mrow > 0.5, e, _NEG)
    m = jnp.max(e, axis=1, keepdims=True)
    p = jnp.exp(e - m)
    s = jnp.sum(p, axis=1, keepdims=True)
    a = p / s
    w_row = jnp.sum(a, axis=0, keepdims=True)

    w8 = jnp.broadcast_to(w_row, (8, L2_))
    wT = jnp.transpose(w8)
    w_col = wT[:, 0:1]

    CH = 256
    TL = 128
    DIAG = CH // TL
    k_loc0 = lax.broadcasted_iota(jnp.int32, (CH, TL), 1)
    j_loc = lax.broadcasted_iota(jnp.int32, (CH, TL), 0)
    less_d = [(k_loc0 + d * TL) < j_loc for d in range(DIAG)]
    r_iota = lax.broadcasted_iota(jnp.int32, (CH, K_), 1)
    jg = lax.broadcasted_iota(jnp.int32, (CH, K_), 0)
    idx_sum = jnp.zeros((1, K_), jnp.int32)
    msk_sum = jnp.zeros((1, K_), jnp.float32)
    for c in range(L2_ // CH):
        wj = w_col[c * CH:(c + 1) * CH, :]
        acc = jnp.zeros((CH, TL), jnp.int32)
        for t in range(L2_ // TL):
            wk = w_row[:, t * TL:(t + 1) * TL]
            if t < c * DIAG:
                hit = (wk >= wj)
            elif t >= (c + 1) * DIAG:
                hit = (wk > wj)
            else:
                hit = (wk > wj) | ((wk == wj) & less_d[t - c * DIAG])
            acc = acc + hit.astype(jnp.int32)
        cnt = jnp.sum(acc, axis=1, keepdims=True)
        oh = (cnt == r_iota)
        idx_sum = idx_sum + jnp.sum(jnp.where(oh, c * CH + jg, 0),
                                    axis=0, keepdims=True)
        mj = mcol[c * CH:(c + 1) * CH, :]
        msk_sum = msk_sum + jnp.sum(jnp.where(oh, mj, 0.0),
                                    axis=0, keepdims=True)

    gidx_ref[...] = (idx_sum + b * L2_).reshape(1, 1, K_)
    xmask_ref[...] = msk_sum.reshape(1, 1, K_)


def _tc_topk(g_r1, g_r2, mask_row, off=0, nb=B_, interpret=False):
    return pl.pallas_call(
        functools.partial(_tc_body, off),
        grid=(nb,),
        in_specs=[
            pl.BlockSpec((1, L1_, D_), lambda b: (b + off, 0, 0)),
            pl.BlockSpec((1, L2_, D_), lambda b: (b + off, 0, 0)),
            pl.BlockSpec((1, 1, L2_), lambda b: (b + off, 0, 0)),
        ],
        out_specs=[
            pl.BlockSpec((1, 1, K_), lambda b: (b, 0, 0)),
            pl.BlockSpec((1, 1, K_), lambda b: (b, 0, 0)),
        ],
        out_shape=[
            jax.ShapeDtypeStruct((nb, 1, K_), jnp.int32),
            jax.ShapeDtypeStruct((nb, 1, K_), jnp.float32),
        ],
        interpret=interpret,
    )(g_r1, g_r2, mask_row)


_NC, _NS = 2, 16
_NW = _NC * _NS
_ROWS = B_ * K_
_BPW = _ROWS // _NW


@functools.lru_cache(maxsize=4)
def _sc_gather_fn(nrows):
    bpw = nrows // _NW

    half = bpw // 2

    @functools.partial(
        pl.kernel,
        mesh=plsc.VectorSubcoreMesh(core_axis_name="c", subcore_axis_name="s"),
        out_type=jax.ShapeDtypeStruct((nrows, D_), jnp.float32),
        scratch_types=[
            pltpu.VMEM((half,), jnp.int32),
            pltpu.VMEM((half,), jnp.int32),
            pltpu.VMEM((half, D_), jnp.float32),
            pltpu.VMEM((half, D_), jnp.float32),
            pltpu.SemaphoreType.DMA,
            pltpu.SemaphoreType.DMA,
            pltpu.SemaphoreType.DMA,
        ],
    )
    def _sc_gather(table_hbm, idx_hbm, out_hbm,
                   idx0, idx1, rows0, rows1, g0, g1, so):
        wid = lax.axis_index("s") * _NC + lax.axis_index("c")
        base = wid * bpw
        pltpu.sync_copy(idx_hbm.at[pl.ds(base, half)], idx0)
        a0 = pltpu.async_copy(table_hbm.at[idx0], rows0, g0)
        pltpu.sync_copy(idx_hbm.at[pl.ds(base + half, half)], idx1)
        a1 = pltpu.async_copy(table_hbm.at[idx1], rows1, g1)
        a0.wait()
        o0 = pltpu.async_copy(rows0, out_hbm.at[pl.ds(base, half)], so)
        a1.wait()
        o1 = pltpu.async_copy(rows1, out_hbm.at[pl.ds(base + half, half)], so)
        o0.wait()
        o1.wait()

    return _sc_gather


def kernel(g_r1, g_r2, g_r1_mask, g_r2_mask):
    mrow = g_r2_mask.reshape(B_, 1, L2_).astype(jnp.float32)
    table = g_r2.reshape(B_ * L2_, D_)
    gidx, xmask = _tc_topk(g_r1, g_r2, mrow)
    xflat = _sc_gather_fn(_ROWS)(table, gidx.reshape(_ROWS))
    x = xflat.reshape(B_, K_, D_)
    x_mask = (xmask.reshape(B_, 1, 1, K_) > 0.5)
    return (x, x_mask)
